# baseline (speedup 1.0000x reference)
import functools

import numpy as np
import jax
import jax.numpy as jnp
from jax.experimental import pallas as pl

_N = 2048
_K = 1984
_FROZEN = 64
_CHUNK = 128
_NCHUNK = _N // _CHUNK

_F = np.array([[1, 0], [1, 1]], dtype=np.int64)
_B7 = functools.reduce(np.kron, [_F] * 7).astype(np.float32)
_W = np.concatenate(
    [np.zeros((64, 128), np.float32), _B7, np.zeros((64, 128), np.float32)],
    axis=0)


def _polar_block(u_ref, w_ref, o_ref):
    u = u_ref[...].astype(jnp.bfloat16)
    w = w_ref[...]

    def mm(lhs, wmat):
        return jax.lax.dot_general(
            lhs, wmat, (((1,), (0,)), ((), ())),
            preferred_element_type=jnp.float32)

    chunks = [mm(u[:, :_CHUNK], w[_CHUNK:, :])]
    for c in range(1, _NCHUNK - 1):
        chunks.append(mm(u[:, (c - 1) * _CHUNK:(c + 1) * _CHUNK], w))
    chunks.append(mm(u[:, (_NCHUNK - 2) * _CHUNK:_K], w[:_K % _CHUNK + _CHUNK, :]))
    for t in range(4):
        step = 1 << t
        for c in range(_NCHUNK):
            if (c >> t) & 1 == 0:
                chunks[c] = chunks[c] + chunks[c + step]
    y = jnp.concatenate(chunks, axis=1)
    bits = jax.lax.bitcast_convert_type(y + jnp.float32(8388608.0), jnp.int32)
    o_ref[...] = jnp.bitwise_and(bits, 1).astype(jnp.float32)


@jax.jit
def kernel(u):
    bs = u.shape[0]
    block_rows = 512
    grid = bs // block_rows
    w = jnp.asarray(_W, jnp.bfloat16)
    return pl.pallas_call(
        _polar_block,
        grid=(grid,),
        in_specs=[
            pl.BlockSpec((block_rows, _K), lambda i: (i, 0)),
            pl.BlockSpec((2 * _CHUNK, _CHUNK), lambda i: (0, 0)),
        ],
        out_specs=pl.BlockSpec((block_rows, _N), lambda i: (i, 0)),
        out_shape=jax.ShapeDtypeStruct((bs, _N), jnp.float32),
    )(u, w)


# speedup vs baseline: 1.0161x; 1.0161x over previous
import functools

import numpy as np
import jax
import jax.numpy as jnp
from jax.experimental import pallas as pl
from jax.experimental.pallas import tpu as pltpu

_N = 2048
_K = 1984
_FROZEN = 64
_CHUNK = 128
_NCHUNK = _N // _CHUNK

_F = np.array([[1, 0], [1, 1]], dtype=np.int64)
_B7 = functools.reduce(np.kron, [_F] * 7).astype(np.float32)
_W = np.concatenate(
    [np.zeros((64, 128), np.float32), _B7, np.zeros((64, 128), np.float32)],
    axis=0)


def _polar_block(u_ref, w_ref, o_ref):
    u = u_ref[...].astype(jnp.bfloat16)
    w = w_ref[...]

    def mm(lhs, wmat):
        return jax.lax.dot_general(
            lhs, wmat, (((1,), (0,)), ((), ())),
            preferred_element_type=jnp.float32)

    chunks = [mm(u[:, :_CHUNK], w[_CHUNK:, :])]
    for c in range(1, _NCHUNK - 1):
        chunks.append(mm(u[:, (c - 1) * _CHUNK:(c + 1) * _CHUNK], w))
    chunks.append(mm(u[:, (_NCHUNK - 2) * _CHUNK:_K], w[:_K % _CHUNK + _CHUNK, :]))
    for t in range(4):
        step = 1 << t
        for c in range(_NCHUNK):
            if (c >> t) & 1 == 0:
                chunks[c] = chunks[c] + chunks[c + step]
    y = jnp.concatenate(chunks, axis=1)
    bits = jax.lax.bitcast_convert_type(y + jnp.float32(8388608.0), jnp.int32)
    o_ref[...] = jnp.bitwise_and(bits, 1).astype(jnp.float32)


@jax.jit
def kernel(u):
    bs = u.shape[0]
    block_rows = 1024
    grid = bs // block_rows
    w = jnp.asarray(_W, jnp.bfloat16)
    return pl.pallas_call(
        _polar_block,
        grid=(grid,),
        in_specs=[
            pl.BlockSpec((block_rows, _K), lambda i: (i, 0)),
            pl.BlockSpec((2 * _CHUNK, _CHUNK), lambda i: (0, 0)),
        ],
        out_specs=pl.BlockSpec((block_rows, _N), lambda i: (i, 0)),
        out_shape=jax.ShapeDtypeStruct((bs, _N), jnp.float32),
        compiler_params=pltpu.CompilerParams(dimension_semantics=("parallel",)),
    )(u, w)
